# baseline (speedup 1.0000x reference)
import jax
import jax.numpy as jnp
from jax.experimental import pallas as pl

_BB = 128


def _body(x_ref, act_ref, gate_ref, disp_ref):
    xv = x_ref[...]
    t = xv * 2.0
    t2 = t * t
    q = t2 * t2
    gate = 1.0 / (1.0 + q)
    gate_ref[...] = gate
    gm = jnp.where(gate >= 0.5, gate, 0.0)
    a = act_ref[...]
    disp_ref[...] = gm[:, :, None] * a[:, None, :]


def kernel(x, act, batch_inds):
    n, e = x.shape
    d = act.shape[1]
    grid = (n // _BB,)
    gate, disp = pl.pallas_call(
        _body,
        grid=grid,
        in_specs=[
            pl.BlockSpec((_BB, e), lambda i: (i, 0)),
            pl.BlockSpec((_BB, d), lambda i: (i, 0)),
        ],
        out_specs=[
            pl.BlockSpec((_BB, e), lambda i: (i, 0)),
            pl.BlockSpec((_BB, e, d), lambda i: (i, 0, 0)),
        ],
        out_shape=[
            jax.ShapeDtypeStruct((n, e), jnp.float32),
            jax.ShapeDtypeStruct((n, e, d), jnp.float32),
        ],
    )(x, act)
    return gate, disp


# speedup vs baseline: 1.1911x; 1.1911x over previous
import jax
import jax.numpy as jnp
from jax.experimental import pallas as pl

_BB = 512


def _body(x_ref, act_ref, gate_ref, disp_ref):
    xv = x_ref[...]
    t = xv * 2.0
    t2 = t * t
    q = t2 * t2
    gate = 1.0 / (1.0 + q)
    gate_ref[...] = gate
    gm = jnp.where(gate >= 0.5, gate, 0.0)
    a = act_ref[...]
    disp_ref[...] = gm[:, :, None] * a[:, None, :]


def kernel(x, act, batch_inds):
    n, e = x.shape
    d = act.shape[1]
    grid = (n // _BB,)
    gate, disp = pl.pallas_call(
        _body,
        grid=grid,
        in_specs=[
            pl.BlockSpec((_BB, e), lambda i: (i, 0)),
            pl.BlockSpec((_BB, d), lambda i: (i, 0)),
        ],
        out_specs=[
            pl.BlockSpec((_BB, e), lambda i: (i, 0)),
            pl.BlockSpec((_BB, e, d), lambda i: (i, 0, 0)),
        ],
        out_shape=[
            jax.ShapeDtypeStruct((n, e), jnp.float32),
            jax.ShapeDtypeStruct((n, e, d), jnp.float32),
        ],
    )(x, act)
    return gate, disp
